# Initial kernel scaffold; baseline (speedup 1.0000x reference)
import functools

import jax
import jax.numpy as jnp
from jax import lax
from jax.experimental import pallas as pl
from jax.experimental.pallas import tpu as pltpu
from jax.experimental.pallas import tpu_sc as plsc

NUM_FIELDS = 26
VOCAB = 100000
DIM = 32
BATCH = 16384

NC = 2
NS = 16
NW = NC * NS
L = 16

R = BATCH * NUM_FIELDS
RW = R // NW
CHUNK = 416
NCHUNK = RW // CHUNK
SUB = 104
NSUB = CHUNK // SUB

assert RW * NW == R and NCHUNK * CHUNK == RW and NSUB * SUB == CHUNK
assert CHUNK % NUM_FIELDS == 0 and CHUNK % L == 0 and SUB % 8 == 0

_mesh = plsc.VectorSubcoreMesh(core_axis_name="c", subcore_axis_name="s")


@functools.partial(
    pl.kernel,
    out_type=jax.ShapeDtypeStruct((R, DIM), jnp.float32),
    mesh=_mesh,
    scratch_types=[
        pltpu.VMEM((CHUNK,), jnp.int32),
        pltpu.VMEM((CHUNK,), jnp.int32),
        pltpu.VMEM((CHUNK, DIM), jnp.float32),
        pltpu.SemaphoreType.DMA,
    ],
)
def _embed(tab_hbm, idx_hbm, out_hbm, off_v, idx_v, rows_v, sem):
    wid = lax.axis_index("s") * NC + lax.axis_index("c")
    base = wid * RW

    iota = lax.iota(jnp.int32, L)
    for i in range(CHUNK // L):
        pos = iota + (i * L)
        off_v[pl.ds(i * L, L)] = (pos % NUM_FIELDS) * VOCAB

    def chunk_body(g, carry):
        cb = base + g * CHUNK
        pltpu.sync_copy(idx_hbm.at[pl.ds(cb, CHUNK)], idx_v)
        for i in range(CHUNK // L):
            sl = pl.ds(i * L, L)
            idx_v[sl] = idx_v[sl] + off_v[sl]
        cps = [
            pltpu.async_copy(
                tab_hbm.at[idx_v.at[pl.ds(j * SUB, SUB)]],
                rows_v.at[pl.ds(j * SUB, SUB)],
                sem,
            )
            for j in range(NSUB)
        ]
        for cp in cps:
            cp.wait()
        pltpu.sync_copy(rows_v, out_hbm.at[pl.ds(cb, CHUNK)])
        return carry

    lax.fori_loop(0, NCHUNK, chunk_body, 0)


def kernel(encoded_tensor, tables):
    flat_tab = tables.reshape(NUM_FIELDS * VOCAB, DIM)
    flat_idx = encoded_tensor.reshape(R)
    out = _embed(flat_tab, flat_idx)
    return out.reshape(BATCH, NUM_FIELDS * DIM)


# speedup vs baseline: 1.1797x; 1.1797x over previous
import functools

import jax
import jax.numpy as jnp
from jax import lax
from jax.experimental import pallas as pl
from jax.experimental.pallas import tpu as pltpu
from jax.experimental.pallas import tpu_sc as plsc

NUM_FIELDS = 26
VOCAB = 100000
DIM = 32
BATCH = 16384

NC = 2
NS = 16
NW = NC * NS
L = 16

R = BATCH * NUM_FIELDS
RW = R // NW
CHUNK = 416
NCHUNK = RW // CHUNK
SUB = 104
NSUB = CHUNK // SUB

assert RW * NW == R and NCHUNK * CHUNK == RW and NSUB * SUB == CHUNK
assert CHUNK % NUM_FIELDS == 0 and CHUNK % L == 0 and SUB % 8 == 0

_mesh = plsc.VectorSubcoreMesh(core_axis_name="c", subcore_axis_name="s")


@functools.partial(
    pl.kernel,
    out_type=jax.ShapeDtypeStruct((R, DIM), jnp.float32),
    mesh=_mesh,
    scratch_types=[
        pltpu.VMEM((CHUNK,), jnp.int32),
        pltpu.VMEM((CHUNK,), jnp.int32),
        pltpu.VMEM((CHUNK, DIM), jnp.float32),
        pltpu.SemaphoreType.DMA,
    ],
    compiler_params=pltpu.CompilerParams(use_tc_tiling_on_sc=False),
)
def _embed(tab_hbm, idx_hbm, out_hbm, off_v, idx_v, rows_v, sem):
    wid = lax.axis_index("s") * NC + lax.axis_index("c")
    base = wid * RW

    iota = lax.iota(jnp.int32, L)
    for i in range(CHUNK // L):
        pos = iota + (i * L)
        off_v[pl.ds(i * L, L)] = (pos % NUM_FIELDS) * VOCAB

    def chunk_body(g, carry):
        cb = base + g * CHUNK
        pltpu.sync_copy(idx_hbm.at[pl.ds(cb, CHUNK)], idx_v)
        for i in range(CHUNK // L):
            sl = pl.ds(i * L, L)
            idx_v[sl] = idx_v[sl] + off_v[sl]
        cps = [
            pltpu.async_copy(
                tab_hbm.at[idx_v.at[pl.ds(j * SUB, SUB)]],
                rows_v.at[pl.ds(j * SUB, SUB)],
                sem,
            )
            for j in range(NSUB)
        ]
        for cp in cps:
            cp.wait()
        pltpu.sync_copy(rows_v, out_hbm.at[pl.ds(cb, CHUNK)])
        return carry

    lax.fori_loop(0, NCHUNK, chunk_body, 0)


def kernel(encoded_tensor, tables):
    flat_tab = tables.reshape(NUM_FIELDS * VOCAB, DIM)
    flat_idx = encoded_tensor.reshape(R)
    out = _embed(flat_tab, flat_idx)
    return out.reshape(BATCH, NUM_FIELDS * DIM)


# speedup vs baseline: 1.2144x; 1.0294x over previous
import functools

import jax
import jax.numpy as jnp
from jax import lax
from jax.experimental import pallas as pl
from jax.experimental.pallas import tpu as pltpu
from jax.experimental.pallas import tpu_sc as plsc

NUM_FIELDS = 26
VOCAB = 100000
DIM = 32
BATCH = 16384

NC = 2
NS = 16
NW = NC * NS
L = 16

R = BATCH * NUM_FIELDS
RW = R // NW
SUB = 128
K = 4
GROUP = K * SUB
NG = RW // GROUP

assert RW * NW == R and NG * GROUP == RW and NG % 2 == 0
assert GROUP % L == 0 and RW % NUM_FIELDS == 0

_mesh = plsc.VectorSubcoreMesh(core_axis_name="c", subcore_axis_name="s")


@functools.partial(
    pl.kernel,
    out_type=jax.ShapeDtypeStruct((R, DIM), jnp.float32),
    mesh=_mesh,
    scratch_types=[
        pltpu.VMEM((RW,), jnp.int32),
        pltpu.VMEM((2, GROUP, DIM), jnp.float32),
        pltpu.SemaphoreType.DMA,
        pltpu.SemaphoreType.DMA,
        pltpu.SemaphoreType.DMA,
        pltpu.SemaphoreType.DMA,
    ],
    compiler_params=pltpu.CompilerParams(use_tc_tiling_on_sc=False),
)
def _embed(tab_hbm, idx_hbm, out_hbm, idx_v, rows_v, g0, g1, s0, s1):
    wid = lax.axis_index("s") * NC + lax.axis_index("c")
    base = wid * RW
    iota = lax.iota(jnp.int32, L)
    gsem = (g0, g1)
    ssem = (s0, s1)

    pltpu.sync_copy(idx_hbm.at[pl.ds(base, RW)], idx_v)

    def add_offsets_and_fire(g, p):
        gb = g * GROUP
        for s in range(GROUP // L):
            sl = pl.ds(gb + s * L, L)
            pos = (base + gb + s * L) + iota
            idx_v[sl] = idx_v[sl] + lax.rem(pos, NUM_FIELDS) * VOCAB
        for j in range(K):
            pltpu.async_copy(
                tab_hbm.at[idx_v.at[pl.ds(gb + j * SUB, SUB)]],
                rows_v.at[p, pl.ds(j * SUB, SUB)],
                gsem[p],
            )

    def drain_gathers(p):
        pltpu.make_async_copy(
            out_hbm.at[pl.ds(0, GROUP)], rows_v.at[p], gsem[p]
        ).wait()

    def fire_store(g, p):
        pltpu.async_copy(
            rows_v.at[p], out_hbm.at[pl.ds(base + g * GROUP, GROUP)], ssem[p]
        )

    def drain_store(p):
        pltpu.make_async_copy(
            rows_v.at[p], out_hbm.at[pl.ds(0, GROUP)], ssem[p]
        ).wait()

    add_offsets_and_fire(0, 0)
    add_offsets_and_fire(1, 1)
    drain_gathers(0)
    fire_store(0, 0)

    def body(u, carry):
        g = 2 * u + 2
        drain_store(0)
        add_offsets_and_fire(g, 0)
        drain_gathers(1)
        fire_store(g - 1, 1)
        drain_store(1)
        add_offsets_and_fire(g + 1, 1)
        drain_gathers(0)
        fire_store(g, 0)
        return carry

    lax.fori_loop(0, (NG - 2) // 2, body, 0)

    drain_gathers(1)
    fire_store(NG - 1, 1)
    drain_store(0)
    drain_store(1)


def kernel(encoded_tensor, tables):
    flat_tab = tables.reshape(NUM_FIELDS * VOCAB, DIM)
    flat_idx = encoded_tensor.reshape(R)
    out = _embed(flat_tab, flat_idx)
    return out.reshape(BATCH, NUM_FIELDS * DIM)


# speedup vs baseline: 3.6757x; 3.0267x over previous
import functools

import jax
import jax.numpy as jnp
from jax import lax
from jax.experimental import pallas as pl
from jax.experimental.pallas import tpu as pltpu
from jax.experimental.pallas import tpu_sc as plsc

NUM_FIELDS = 26
VOCAB = 100000
DIM = 32
BATCH = 16384

NC = 2
NS = 16
NW = NC * NS
L = 16

NCOL = NUM_FIELDS * DIM
CPW = NCOL // NW
ICH = 4096
NICH = BATCH // ICH

assert CPW * NW == NCOL and NICH * ICH == BATCH and ICH % L == 0

_mesh = plsc.VectorSubcoreMesh(core_axis_name="c", subcore_axis_name="s")


@functools.partial(
    pl.kernel,
    out_type=jax.ShapeDtypeStruct((NCOL, BATCH), jnp.float32),
    mesh=_mesh,
    scratch_types=[
        pltpu.VMEM((VOCAB,), jnp.float32),
        pltpu.VMEM((2, ICH), jnp.int32),
        pltpu.VMEM((2, ICH), jnp.float32),
        pltpu.SemaphoreType.DMA,
        pltpu.SemaphoreType.DMA,
    ],
    compiler_params=pltpu.CompilerParams(
        use_tc_tiling_on_sc=True, needs_layout_passes=False
    ),
)
def _embed(tab_hbm, enc_hbm, out_hbm, col_v, idx_v, res_v, isem, osem):
    wid = lax.axis_index("s") * NC + lax.axis_index("c")

    def col_body(j, carry):
        r = wid * CPW + j
        f = r // DIM
        c = lax.rem(r, DIM)
        pltpu.sync_copy(tab_hbm.at[f, c], col_v)

        def chunk_body(k, carry2):
            p = lax.rem(k, 2)
            pltpu.sync_copy(enc_hbm.at[f, pl.ds(k * ICH, ICH)], idx_v.at[p])
            for i in range(ICH // L):
                sl = pl.ds(i * L, L)
                iv = idx_v[p, sl]
                res_v[p, sl] = plsc.load_gather(col_v, [iv])
            pltpu.sync_copy(res_v.at[p], out_hbm.at[r, pl.ds(k * ICH, ICH)])
            return carry2

        lax.fori_loop(0, NICH, chunk_body, 0)
        return carry

    lax.fori_loop(0, CPW, col_body, 0)


def kernel(encoded_tensor, tables):
    tab_t = tables.transpose(0, 2, 1)
    enc_t = encoded_tensor.transpose(1, 0)
    out = _embed(tab_t, enc_t)
    return out.transpose(1, 0)


# speedup vs baseline: 5.1475x; 1.4004x over previous
import functools

import jax
import jax.numpy as jnp
from jax import lax
from jax.experimental import pallas as pl
from jax.experimental.pallas import tpu as pltpu
from jax.experimental.pallas import tpu_sc as plsc

NUM_FIELDS = 26
VOCAB = 100000
DIM = 32
BATCH = 16384

NC = 2
NS = 16
NW = NC * NS
L = 16

NCOL = NUM_FIELDS * DIM
CPW = NCOL // NW
OCH = 4096
NOCH = BATCH // OCH

assert CPW * NW == NCOL and NOCH * OCH == BATCH and OCH % L == 0

_mesh = plsc.VectorSubcoreMesh(core_axis_name="c", subcore_axis_name="s")


@functools.partial(
    pl.kernel,
    out_type=jax.ShapeDtypeStruct((NCOL, BATCH), jnp.float32),
    mesh=_mesh,
    scratch_types=[
        pltpu.VMEM((VOCAB,), jnp.float32),
        pltpu.VMEM((BATCH,), jnp.int32),
        pltpu.VMEM((2, OCH), jnp.float32),
        pltpu.SemaphoreType.DMA,
        pltpu.SemaphoreType.DMA,
    ],
    compiler_params=pltpu.CompilerParams(
        use_tc_tiling_on_sc=True, needs_layout_passes=False
    ),
)
def _embed(tab_hbm, enc_hbm, out_hbm, col_v, idx_v, res_v, o0, o1):
    wid = lax.axis_index("s") * NC + lax.axis_index("c")
    osem = (o0, o1)

    def drain(p, r):
        pltpu.make_async_copy(
            res_v.at[p], out_hbm.at[r, pl.ds(0, OCH)], osem[p]
        ).wait()

    def col_body(j, f_prev):
        r = wid * CPW + j
        f = r // DIM
        c = lax.rem(r, DIM)

        @pl.when(f != f_prev)
        def _():
            pltpu.sync_copy(enc_hbm.at[f], idx_v)

        pltpu.sync_copy(tab_hbm.at[f, c], col_v)

        for k in range(NOCH):
            p = k % 2
            m = j * NOCH + k

            @pl.when(m >= 2)
            def _():
                drain(p, r)

            for i in range(OCH // L):
                sl = pl.ds(k * OCH + i * L, L)
                res_v[p, pl.ds(i * L, L)] = plsc.load_gather(
                    col_v, [idx_v[sl]]
                )
            pltpu.async_copy(
                res_v.at[p], out_hbm.at[r, pl.ds(k * OCH, OCH)], osem[p]
            )
        return f

    lax.fori_loop(0, CPW, col_body, -1)
    drain(0, 0)
    drain(1, 0)


def kernel(encoded_tensor, tables):
    tab_t = tables.transpose(0, 2, 1)
    enc_t = encoded_tensor.transpose(1, 0)
    out = _embed(tab_t, enc_t)
    return out.transpose(1, 0)


# speedup vs baseline: 6.9363x; 1.3475x over previous
import functools

import jax
import jax.numpy as jnp
from jax import lax
from jax.experimental import pallas as pl
from jax.experimental.pallas import tpu as pltpu
from jax.experimental.pallas import tpu_sc as plsc

NUM_FIELDS = 26
VOCAB = 100000
DIM = 32
BATCH = 16384

NC = 2
NS = 16
NW = NC * NS
L = 16

NCOL = NUM_FIELDS * DIM
CPW = NCOL // NW
OCH = 4096
NOCH = BATCH // OCH

assert CPW * NW == NCOL and NOCH * OCH == BATCH and OCH % L == 0

_mesh = plsc.VectorSubcoreMesh(core_axis_name="c", subcore_axis_name="s")


@functools.partial(
    pl.kernel,
    out_type=jax.ShapeDtypeStruct((NCOL, BATCH), jnp.float32),
    mesh=_mesh,
    scratch_types=[
        pltpu.VMEM((VOCAB,), jnp.float32),
        pltpu.VMEM((BATCH,), jnp.int32),
        pltpu.VMEM((2, OCH), jnp.float32),
        pltpu.SemaphoreType.DMA,
        pltpu.SemaphoreType.DMA,
    ],
    compiler_params=pltpu.CompilerParams(
        use_tc_tiling_on_sc=True, needs_layout_passes=False
    ),
)
def _embed(tab_hbm, enc_hbm, out_hbm, col_v, idx_v, res_v, o0, o1):
    wid = lax.axis_index("s") * NC + lax.axis_index("c")
    osem = (o0, o1)

    def drain(p, r):
        pltpu.make_async_copy(
            res_v.at[p], out_hbm.at[r, pl.ds(0, OCH)], osem[p]
        ).wait()

    def col_body(j, f_prev):
        r = wid * CPW + j
        f = r // DIM
        c = lax.rem(r, DIM)

        @pl.when(f != f_prev)
        def _():
            pltpu.sync_copy(enc_hbm.at[f], idx_v)

        pltpu.sync_copy(tab_hbm.at[f, c], col_v)

        for k in range(NOCH):
            p = k % 2
            m = j * NOCH + k

            @pl.when(m >= 2)
            def _():
                drain(p, r)

            U = 8
            for i in range(0, OCH // L, U):
                gs = [
                    plsc.load_gather(
                        col_v, [idx_v[pl.ds(k * OCH + (i + u) * L, L)]]
                    )
                    for u in range(U)
                ]
                for u in range(U):
                    res_v[p, pl.ds((i + u) * L, L)] = gs[u]
            pltpu.async_copy(
                res_v.at[p], out_hbm.at[r, pl.ds(k * OCH, OCH)], osem[p]
            )
        return f

    lax.fori_loop(0, CPW, col_body, -1)
    drain(0, 0)
    drain(1, 0)


def kernel(encoded_tensor, tables):
    tab_t = tables.transpose(0, 2, 1)
    enc_t = encoded_tensor.transpose(1, 0)
    out = _embed(tab_t, enc_t)
    return out.transpose(1, 0)


# speedup vs baseline: 7.1818x; 1.0354x over previous
import functools

import jax
import jax.numpy as jnp
from jax import lax
from jax.experimental import pallas as pl
from jax.experimental.pallas import tpu as pltpu
from jax.experimental.pallas import tpu_sc as plsc

NUM_FIELDS = 26
VOCAB = 100000
DIM = 32
BATCH = 16384

NC = 2
NS = 16
NW = NC * NS
L = 16

NCOL = NUM_FIELDS * DIM
CPW = NCOL // NW
OCH = 4096
NOCH = BATCH // OCH

assert CPW * NW == NCOL and NOCH * OCH == BATCH and OCH % L == 0

_mesh = plsc.VectorSubcoreMesh(core_axis_name="c", subcore_axis_name="s")


@functools.partial(
    pl.kernel,
    out_type=jax.ShapeDtypeStruct((NCOL, BATCH), jnp.float32),
    mesh=_mesh,
    scratch_types=[
        pltpu.VMEM((VOCAB,), jnp.float32),
        pltpu.VMEM((BATCH,), jnp.int32),
        pltpu.VMEM((2, OCH), jnp.float32),
        pltpu.SemaphoreType.DMA,
        pltpu.SemaphoreType.DMA,
        pltpu.SemaphoreType.DMA,
    ],
    compiler_params=pltpu.CompilerParams(
        use_tc_tiling_on_sc=True, needs_layout_passes=False
    ),
)
def _embed(tab_hbm, enc_hbm, out_hbm, col_v, idx_v, res_v, o0, o1, csem):
    wid = lax.axis_index("s") * NC + lax.axis_index("c")
    osem = (o0, o1)

    def fire_col(j):
        r = wid * CPW + j
        pltpu.async_copy(tab_hbm.at[r // DIM, lax.rem(r, DIM)], col_v, csem)

    def drain(p, r):
        pltpu.make_async_copy(
            res_v.at[p], out_hbm.at[r, pl.ds(0, OCH)], osem[p]
        ).wait()

    def col_body(j, f_prev):
        r = wid * CPW + j
        f = r // DIM
        c = lax.rem(r, DIM)

        @pl.when(f != f_prev)
        def _():
            pltpu.sync_copy(enc_hbm.at[f], idx_v)

        pltpu.make_async_copy(tab_hbm.at[0, 0], col_v, csem).wait()

        for k in range(NOCH):
            p = k % 2
            m = j * NOCH + k

            @pl.when(m >= 2)
            def _():
                drain(p, r)

            U = 8
            for i in range(0, OCH // L, U):
                gs = [
                    plsc.load_gather(
                        col_v, [idx_v[pl.ds(k * OCH + (i + u) * L, L)]]
                    )
                    for u in range(U)
                ]
                for u in range(U):
                    res_v[p, pl.ds((i + u) * L, L)] = gs[u]
            if k == NOCH - 1:
                @pl.when(j + 1 < CPW)
                def _():
                    fire_col(j + 1)
            pltpu.async_copy(
                res_v.at[p], out_hbm.at[r, pl.ds(k * OCH, OCH)], osem[p]
            )
        return f

    fire_col(0)
    lax.fori_loop(0, CPW, col_body, -1)
    drain(0, 0)
    drain(1, 0)


def kernel(encoded_tensor, tables):
    tab_t = tables.transpose(0, 2, 1)
    enc_t = encoded_tensor.transpose(1, 0)
    out = _embed(tab_t, enc_t)
    return out.transpose(1, 0)
